# baseline (speedup 1.0000x reference)
import functools

import jax
import jax.numpy as jnp
from jax import lax
from jax.experimental import pallas as pl
from jax.experimental.pallas import tpu as pltpu
from jax.experimental.pallas import tpu_sc as plsc

N = 10000
E = 320000
D = 128
D_EDGE = 16
PROJ = 64
EPS = 1e-3

NC = 2
NS = 16
NW = NC * NS
EW = E // NW
CA = 80
NCHA = EW // CA
CB = 40
NCHB = EW // CB
RPT = 624
_F32 = jnp.float32

_sc_mesh = plsc.VectorSubcoreMesh(core_axis_name="c", subcore_axis_name="s")


def _sa_body(psrc, pdst, t, srci, dsti, epre_out, stats_out,
             idxs, idxd, ba0, ba1, bb0, bb1, bt0, bt1, bo0, bo1, accv,
             sg0, sg1, st0, st1, so0, so1):
    ba = (ba0, ba1)
    bb = (bb0, bb1)
    bt = (bt0, bt1)
    bo = (bo0, bo1)
    sg = (sg0, sg1)
    st = (st0, st1)
    so = (so0, so1)
    cid = lax.axis_index("c")
    sid = lax.axis_index("s")
    wid = sid * NC + cid

    pltpu.sync_copy(srci.at[wid], idxs)
    pltpu.sync_copy(dsti.at[wid], idxd)

    def issue_in(j, s):
        pltpu.async_copy(psrc.at[idxs.at[pl.ds(j * CA, CA)]], ba[s], sg[s])
        pltpu.async_copy(pdst.at[idxd.at[pl.ds(j * CA, CA)]], bb[s], sg[s])
        pltpu.async_copy(t.at[pl.ds(wid * EW + j * CA, CA)], bt[s], st[s])

    def wait_in(s):
        pltpu.make_async_copy(psrc.at[pl.ds(0, CA)], ba[s], sg[s]).wait()
        pltpu.make_async_copy(psrc.at[pl.ds(0, CA)], bb[s], sg[s]).wait()
        pltpu.make_async_copy(t.at[pl.ds(0, CA)], bt[s], st[s]).wait()

    def issue_out(j, s):
        pltpu.async_copy(bo[s], epre_out.at[pl.ds(wid * EW + j * CA, CA)], so[s])

    def wait_out(s):
        pltpu.make_async_copy(bo[s], epre_out.at[pl.ds(0, CA)], so[s]).wait()

    def compute(s, acc):
        def row(r, acc_in):
            accs = list(acc_in)
            for c in range(8):
                sl = pl.ds(c * 16, 16)
                x = ba[s][r, sl] + bb[s][r, sl] + bt[s][r, sl]
                x = jnp.maximum(x, 0.0)
                bo[s][r, sl] = x
                accs[c] = accs[c] + x
                accs[8 + c] = accs[8 + c] + x * x
            return tuple(accs)
        return lax.fori_loop(0, CA, row, acc)

    issue_in(0, 0)
    zero = jnp.zeros((16,), _F32)

    def body(it, acc):
        j = it * 2
        issue_in(j + 1, 1)
        wait_in(0)

        @pl.when(it > 0)
        def _():
            wait_out(0)

        acc = compute(0, acc)
        issue_out(j, 0)

        issue_in(j + 2, 0)
        wait_in(1)

        @pl.when(it > 0)
        def _():
            wait_out(1)

        acc = compute(1, acc)
        issue_out(j + 1, 1)
        return acc

    acc = lax.fori_loop(0, (NCHA - 1) // 2, body, tuple([zero] * 16))
    wait_in(0)
    wait_out(0)
    acc = compute(0, acc)
    issue_out(NCHA - 1, 0)
    wait_out(0)
    wait_out(1)

    def zacc(r, carry):
        for c in range(16):
            accv[r, pl.ds(c * 16, 16)] = zero
        return carry

    lax.fori_loop(0, 8, zacc, 0)
    for c in range(8):
        accv[0, pl.ds(c * 16, 16)] = acc[c]
        accv[0, pl.ds(128 + c * 16, 16)] = acc[8 + c]
    pltpu.sync_copy(accv, stats_out.at[wid])


def _make_sa():
    return pl.kernel(
        _sa_body,
        out_type=[
            jax.ShapeDtypeStruct((E, D), _F32),
            jax.ShapeDtypeStruct((NW, 8, 2 * D), _F32),
        ],
        mesh=_sc_mesh,
        scratch_types=[
            pltpu.VMEM((EW,), jnp.int32),
            pltpu.VMEM((EW,), jnp.int32),
        ] + [pltpu.VMEM((CA, D), _F32)] * 8 + [
            pltpu.VMEM((8, 2 * D), _F32),
        ] + [pltpu.SemaphoreType.DMA] * 6,
    )


def _sb_body(epre, aggi, ss, agg_out,
             idx0, idx1, idx2, idx3, bi0, bi1, bi2, bi3, ssv, agg_s,
             si0, si1, si2, si3, so0, so1, so2, so3):
    idx = (idx0, idx1, idx2, idx3)
    bi = (bi0, bi1, bi2, bi3)
    si = (si0, si1, si2, si3)
    so = (so0, so1, so2, so3)
    cid = lax.axis_index("c")
    sid = lax.axis_index("s")
    wid = sid * NC + cid

    def zrow(r, carry):
        for c in range(8):
            bi0[r, pl.ds(c * 16, 16)] = jnp.zeros((16,), _F32)
        return carry

    lax.fori_loop(0, CB, zrow, 0)
    r0 = sid * RPT
    nfull = RPT // CB
    rem = RPT - nfull * CB
    for k in range(nfull):
        pltpu.sync_copy(bi0, agg_s.at[pl.ds(r0 + k * CB, CB)])
    pltpu.sync_copy(bi0.at[pl.ds(0, rem)], agg_s.at[pl.ds(r0 + nfull * CB, rem)])

    @pl.when(sid == NS - 1)
    def _zero_tail():
        pltpu.sync_copy(bi0.at[pl.ds(0, N - NS * RPT)],
                        agg_s.at[pl.ds(NS * RPT, N - NS * RPT)])

    plsc.subcore_barrier()

    pltpu.sync_copy(ss, ssv)
    scale = [ssv[0, pl.ds(c * 16, 16)] for c in range(8)]
    shift = [ssv[1, pl.ds(c * 16, 16)] for c in range(8)]

    def issue_in(j, s):
        pltpu.async_copy(epre.at[pl.ds(wid * EW + j * CB, CB)], bi[s], si[s])
        pltpu.async_copy(aggi.at[wid, j], idx[s], si[s])

    def wait_in(s):
        pltpu.make_async_copy(epre.at[pl.ds(0, CB)], bi[s], si[s]).wait()
        pltpu.make_async_copy(aggi.at[0, 0], idx[s], si[s]).wait()

    def issue_out(s):
        pltpu.async_copy(bi[s], agg_s.at[idx[s]], so[s], add=True)

    def wait_out(s):
        pltpu.make_async_copy(epre.at[pl.ds(0, CB)], bi[s], so[s]).wait()

    def compute(s):
        def row(r, rc):
            for c in range(8):
                sl = pl.ds(c * 16, 16)
                bi[s][r, sl] = jnp.maximum(bi[s][r, sl] * scale[c] + shift[c], 0.0)
            return rc
        lax.fori_loop(0, CB, row, 0)

    issue_in(0, 0)
    issue_in(1, 1)
    _NB = (NCHB // 4) * 4

    def body(it, carry):
        j = it * 4
        for k in range(4):
            nxt = (k + 2) % 4
            wait_in(k)
            compute(k)
            issue_out(k)
            if k < 2:
                @pl.when(it > 0)
                def _():
                    wait_out(nxt)
            else:
                wait_out(nxt)

            @pl.when(j + k + 2 < NCHB)
            def _():
                issue_in(j + k + 2, nxt)
        return carry

    lax.fori_loop(0, _NB // 4, body, 0)
    wait_in(0)
    compute(0)
    issue_out(0)
    wait_out(2)
    wait_in(1)
    compute(1)
    issue_out(1)
    wait_out(3)
    wait_out(0)
    wait_out(1)
    plsc.subcore_barrier()
    pltpu.sync_copy(agg_s.at[pl.ds(r0, RPT)], agg_out.at[cid, pl.ds(r0, RPT)])

    @pl.when(sid == NS - 1)
    def _flush_tail():
        pltpu.sync_copy(agg_s.at[pl.ds(NS * RPT, N - NS * RPT)],
                        agg_out.at[cid, pl.ds(NS * RPT, N - NS * RPT)])


def _make_sb():
    return pl.kernel(
        _sb_body,
        out_type=[jax.ShapeDtypeStruct((NC, N, D), _F32)],
        mesh=_sc_mesh,
        scratch_types=[pltpu.VMEM((CB,), jnp.int32)] * 4
        + [pltpu.VMEM((CB, D), _F32)] * 4 + [
            pltpu.VMEM((2, D), _F32),
            pltpu.VMEM_SHARED((N, D), _F32),
        ] + [pltpu.SemaphoreType.DMA] * 8,
    )


def _prep_body(h, ws, wd, ps, pd):
    x = h[...]
    ps[...] = jnp.dot(x, ws[...], preferred_element_type=_F32)
    pd[...] = jnp.dot(x, wd[...], preferred_element_type=_F32)


def _prep(h, ws, wd):
    return pl.pallas_call(
        _prep_body,
        out_shape=[jax.ShapeDtypeStruct((N, D), _F32)] * 2,
    )(h, ws, wd)


_BE = 8000
_BE0 = 10000


def _t0_body(e, w, b, o):
    o[...] = jnp.dot(e[...], w[...], preferred_element_type=_F32) + b[...]


def _edge_t0(e, w, b):
    k = e.shape[1]
    return pl.pallas_call(
        _t0_body,
        grid=(E // _BE0,),
        in_specs=[
            pl.BlockSpec((_BE0, k), lambda i: (i, 0)),
            pl.BlockSpec((k, D), lambda i: (0, 0)),
            pl.BlockSpec((1, D), lambda i: (0, 0)),
        ],
        out_specs=pl.BlockSpec((_BE0, D), lambda i: (i, 0)),
        out_shape=jax.ShapeDtypeStruct((E, D), _F32),
    )(e, w, b)


def _t_body(epre, ss, w, b, o):
    e = jnp.maximum(epre[...] * ss[0:1, :] + ss[1:2, :], 0.0)
    o[...] = jnp.dot(e, w[...], preferred_element_type=_F32) + b[...]


def _edge_t(epre, ss, w, b):
    return pl.pallas_call(
        _t_body,
        grid=(E // _BE,),
        in_specs=[
            pl.BlockSpec((_BE, D), lambda i: (i, 0)),
            pl.BlockSpec((2, D), lambda i: (0, 0)),
            pl.BlockSpec((D, D), lambda i: (0, 0)),
            pl.BlockSpec((1, D), lambda i: (0, 0)),
        ],
        out_specs=pl.BlockSpec((_BE, D), lambda i: (i, 0)),
        out_shape=jax.ShapeDtypeStruct((E, D), _F32),
    )(epre, ss, w, b)


def _stats_body(parts, g, b, o):
    p = parts[:, 0, :]
    s = jnp.sum(p[:, :D], axis=0, keepdims=True) / E
    q = jnp.sum(p[:, D:], axis=0, keepdims=True) / E
    var = q - s * s
    scale = g[...] * lax.rsqrt(var + EPS)
    o[...] = jnp.concatenate([scale, b[...] - s * scale], axis=0)


def _stats_fin(parts, g, b):
    return pl.pallas_call(
        _stats_body,
        out_shape=jax.ShapeDtypeStruct((2, D), _F32),
    )(parts, g.reshape(1, D), b.reshape(1, D))


def _bn(x, g, b):
    m = jnp.mean(x, axis=0, keepdims=True)
    v = jnp.mean((x - m) * (x - m), axis=0, keepdims=True)
    return (x - m) * lax.rsqrt(v + EPS) * g + b


def _node_core(h_ref, agg_ref, w):
    agg = agg_ref[0] + agg_ref[1]
    x = (jnp.dot(h_ref[...], w["wg1a"][...], preferred_element_type=_F32)
         + jnp.dot(agg, w["wg1b"][...], preferred_element_type=_F32)
         + w["bg1"][...])
    x = jnp.maximum(_bn(x, w["g_g1"][...], w["b_g1"][...]), 0.0)
    x = jnp.dot(x, w["wg2"][...], preferred_element_type=_F32) + w["bg2"][...]
    x = jnp.maximum(_bn(x, w["g_g2"][...], w["b_g2"][...]), 0.0)
    x = _bn(x, w["g_nn"][...], w["b_nn"][...])
    return jnp.maximum(
        jnp.dot(x, w["wn"][...], preferred_element_type=_F32) + w["bn"][...], 0.0)


_MID_KEYS = ("wg1a", "wg1b", "bg1", "g_g1", "b_g1", "wg2", "bg2", "g_g2",
             "b_g2", "g_nn", "b_nn", "wn", "bn", "wes", "wed")


def _node_mid_body(h_ref, agg_ref, *refs):
    w = dict(zip(_MID_KEYS, refs[:len(_MID_KEYS)]))
    h_out, ps_out, pd_out = refs[len(_MID_KEYS):]
    hn = _node_core(h_ref, agg_ref, w)
    h_out[...] = hn
    ps_out[...] = jnp.dot(hn, w["wes"][...], preferred_element_type=_F32)
    pd_out[...] = jnp.dot(hn, w["wed"][...], preferred_element_type=_F32)


def _node_mid(h, agg, lw, we_next):
    args = lw + [we_next[:D], we_next[D:2 * D]]
    return pl.pallas_call(
        _node_mid_body,
        out_shape=[jax.ShapeDtypeStruct((N, D), _F32)] * 3,
    )(h, agg, *args)


_LAST_KEYS = _MID_KEYS[:13] + ("wp1", "bp1", "wp2", "bp2")


def _node_last_body(h_ref, agg_ref, *refs):
    w = dict(zip(_LAST_KEYS, refs[:len(_LAST_KEYS)]))
    out = refs[len(_LAST_KEYS)]
    hn = _node_core(h_ref, agg_ref, w)
    g = jnp.mean(hn, axis=0, keepdims=True)
    g = jnp.maximum(
        jnp.dot(g, w["wp1"][...], preferred_element_type=_F32) + w["bp1"][...], 0.0)
    out[...] = jnp.maximum(
        jnp.dot(g, w["wp2"][...], preferred_element_type=_F32) + w["bp2"][...], 0.0)


def _node_last(h, agg, lw, wp1, bp1, wp2, bp2):
    args = lw + [wp1, bp1.reshape(1, PROJ * 2), wp2, bp2.reshape(1, PROJ)]
    return pl.pallas_call(
        _node_last_body,
        out_shape=jax.ShapeDtypeStruct((1, PROJ), _F32),
    )(h, agg, *args)


def _layer_weights(p, i):
    wg1 = p["l%d_Wg1" % i]
    return [
        wg1[:D], wg1[D:], p["l%d_bg1" % i].reshape(1, D),
        p["l%d_g_g1" % i].reshape(1, D), p["l%d_b_g1" % i].reshape(1, D),
        p["l%d_Wg2" % i], p["l%d_bg2" % i].reshape(1, D),
        p["l%d_g_g2" % i].reshape(1, D), p["l%d_b_g2" % i].reshape(1, D),
        p["l%d_g_nn" % i].reshape(1, D), p["l%d_b_nn" % i].reshape(1, D),
        p["l%d_Wn" % i], p["l%d_bn" % i].reshape(1, D),
    ]


def kernel(node_attributes, edge_attributes, edge_indices, edge_indices_reverse, params):
    nodes = node_attributes[0]
    edges = edge_attributes[0]
    src = edge_indices[0, :, 0].reshape(NW, EW)
    dst = edge_indices[0, :, 1].reshape(NW, EW)
    agg_idx = edge_indices_reverse[0].reshape(NW, NCHB, CB)
    p = params

    sa = _make_sa()
    sb = _make_sb()

    h = nodes
    ps, pd = _prep(h, p["l0_We"][:D], p["l0_We"][D:2 * D])
    t = _edge_t0(edges, p["l0_We"][2 * D:], p["l0_be"].reshape(1, D))
    for i in range(3):
        epre, stats = sa(ps, pd, t, src, dst)
        ss = _stats_fin(stats, p["l%d_g_en" % i], p["l%d_b_en" % i])
        (agg,) = sb(epre, agg_idx, ss)
        if i < 2:
            we_n = p["l%d_We" % (i + 1)]
            t = _edge_t(epre, ss, we_n[2 * D:], p["l%d_be" % (i + 1)].reshape(1, D))
            h, ps, pd = _node_mid(h, agg, _layer_weights(p, i), we_n)
        else:
            emb = _node_last(h, agg, _layer_weights(p, i),
                             p["Wp1"], p["bp1"], p["Wp2"], p["bp2"])

    view = jnp.broadcast_to(emb[:, None, :], (1, 2, PROJ))
    return (emb, view)


# speedup vs baseline: 1.0216x; 1.0216x over previous
import functools

import jax
import jax.numpy as jnp
from jax import lax
from jax.experimental import pallas as pl
from jax.experimental.pallas import tpu as pltpu
from jax.experimental.pallas import tpu_sc as plsc

N = 10000
E = 320000
D = 128
D_EDGE = 16
PROJ = 64
EPS = 1e-3

NC = 2
NS = 16
NW = NC * NS
EW = E // NW
CA = 80
NCHA = EW // CA
CB = 40
NCHB = EW // CB
RPT = 624
_F32 = jnp.float32

_sc_mesh = plsc.VectorSubcoreMesh(core_axis_name="c", subcore_axis_name="s")


def _sa_body(psrc, pdst, t, srci, dsti, epre_out, stats_out,
             idxs, idxd, ba0, ba1, bb0, bb1, bt0, bt1, bo0, bo1, accv,
             sg0, sg1, st0, st1, so0, so1):
    ba = (ba0, ba1)
    bb = (bb0, bb1)
    bt = (bt0, bt1)
    bo = (bo0, bo1)
    sg = (sg0, sg1)
    st = (st0, st1)
    so = (so0, so1)
    cid = lax.axis_index("c")
    sid = lax.axis_index("s")
    wid = sid * NC + cid

    pltpu.sync_copy(srci.at[wid], idxs)
    pltpu.sync_copy(dsti.at[wid], idxd)

    def issue_in(j, s):
        pltpu.async_copy(psrc.at[idxs.at[pl.ds(j * CA, CA)]], ba[s], sg[s])
        pltpu.async_copy(pdst.at[idxd.at[pl.ds(j * CA, CA)]], bb[s], sg[s])
        pltpu.async_copy(t.at[pl.ds(wid * EW + j * CA, CA)], bt[s], st[s])

    def wait_in(s):
        pltpu.make_async_copy(psrc.at[pl.ds(0, CA)], ba[s], sg[s]).wait()
        pltpu.make_async_copy(psrc.at[pl.ds(0, CA)], bb[s], sg[s]).wait()
        pltpu.make_async_copy(t.at[pl.ds(0, CA)], bt[s], st[s]).wait()

    def issue_out(j, s):
        pltpu.async_copy(bo[s], epre_out.at[pl.ds(wid * EW + j * CA, CA)], so[s])

    def wait_out(s):
        pltpu.make_async_copy(bo[s], epre_out.at[pl.ds(0, CA)], so[s]).wait()

    def compute(s, acc):
        def row(r, acc_in):
            accs = list(acc_in)
            for c in range(8):
                sl = pl.ds(c * 16, 16)
                x = ba[s][r, sl] + bb[s][r, sl] + bt[s][r, sl]
                x = jnp.maximum(x, 0.0)
                bo[s][r, sl] = x
                accs[c] = accs[c] + x
                accs[8 + c] = accs[8 + c] + x * x
            return tuple(accs)
        return lax.fori_loop(0, CA, row, acc)

    issue_in(0, 0)
    zero = jnp.zeros((16,), _F32)

    def body(it, acc):
        j = it * 2
        issue_in(j + 1, 1)
        wait_in(0)

        @pl.when(it > 0)
        def _():
            wait_out(0)

        acc = compute(0, acc)
        issue_out(j, 0)

        issue_in(j + 2, 0)
        wait_in(1)

        @pl.when(it > 0)
        def _():
            wait_out(1)

        acc = compute(1, acc)
        issue_out(j + 1, 1)
        return acc

    acc = lax.fori_loop(0, (NCHA - 1) // 2, body, tuple([zero] * 16))
    wait_in(0)
    wait_out(0)
    acc = compute(0, acc)
    issue_out(NCHA - 1, 0)
    wait_out(0)
    wait_out(1)

    def zacc(r, carry):
        for c in range(16):
            accv[r, pl.ds(c * 16, 16)] = zero
        return carry

    lax.fori_loop(0, 8, zacc, 0)
    for c in range(8):
        accv[0, pl.ds(c * 16, 16)] = acc[c]
        accv[0, pl.ds(128 + c * 16, 16)] = acc[8 + c]
    pltpu.sync_copy(accv, stats_out.at[wid])


def _make_sa():
    return pl.kernel(
        _sa_body,
        out_type=[
            jax.ShapeDtypeStruct((E, D), _F32),
            jax.ShapeDtypeStruct((NW, 8, 2 * D), _F32),
        ],
        mesh=_sc_mesh,
        scratch_types=[
            pltpu.VMEM((EW,), jnp.int32),
            pltpu.VMEM((EW,), jnp.int32),
        ] + [pltpu.VMEM((CA, D), _F32)] * 8 + [
            pltpu.VMEM((8, 2 * D), _F32),
        ] + [pltpu.SemaphoreType.DMA] * 6,
    )


def _sb_body(epre, aggi, ss, agg_out,
             idx0, idx1, idx2, idx3, bi0, bi1, bi2, bi3, ssv, agg_s,
             si0, si1, si2, si3, so0, so1, so2, so3):
    idx = (idx0, idx1, idx2, idx3)
    bi = (bi0, bi1, bi2, bi3)
    si = (si0, si1, si2, si3)
    so = (so0, so1, so2, so3)
    cid = lax.axis_index("c")
    sid = lax.axis_index("s")
    wid = sid * NC + cid

    def zrow(r, carry):
        for c in range(8):
            bi0[r, pl.ds(c * 16, 16)] = jnp.zeros((16,), _F32)
        return carry

    lax.fori_loop(0, CB, zrow, 0)
    r0 = sid * RPT
    nfull = RPT // CB
    rem = RPT - nfull * CB
    for k in range(nfull):
        pltpu.sync_copy(bi0, agg_s.at[pl.ds(r0 + k * CB, CB)])
    pltpu.sync_copy(bi0.at[pl.ds(0, rem)], agg_s.at[pl.ds(r0 + nfull * CB, rem)])

    @pl.when(sid == NS - 1)
    def _zero_tail():
        pltpu.sync_copy(bi0.at[pl.ds(0, N - NS * RPT)],
                        agg_s.at[pl.ds(NS * RPT, N - NS * RPT)])

    plsc.subcore_barrier()

    pltpu.sync_copy(ss, ssv)
    scale = [ssv[0, pl.ds(c * 16, 16)] for c in range(8)]
    shift = [ssv[1, pl.ds(c * 16, 16)] for c in range(8)]

    def issue_in(j, s):
        pltpu.async_copy(epre.at[pl.ds(wid * EW + j * CB, CB)], bi[s], si[s])
        pltpu.async_copy(aggi.at[wid, j], idx[s], si[s])

    def wait_in(s):
        pltpu.make_async_copy(epre.at[pl.ds(0, CB)], bi[s], si[s]).wait()
        pltpu.make_async_copy(aggi.at[0, 0], idx[s], si[s]).wait()

    def issue_out(s):
        pltpu.async_copy(bi[s], agg_s.at[idx[s]], so[s], add=True)

    def wait_out(s):
        pltpu.make_async_copy(epre.at[pl.ds(0, CB)], bi[s], so[s]).wait()

    def compute(s):
        def row(r, rc):
            for c in range(8):
                sl = pl.ds(c * 16, 16)
                bi[s][r, sl] = jnp.maximum(bi[s][r, sl] * scale[c] + shift[c], 0.0)
            return rc
        lax.fori_loop(0, CB, row, 0)

    issue_in(0, 0)
    issue_in(1, 1)
    _NB = (NCHB // 4) * 4

    def body(it, carry):
        j = it * 4
        for k in range(4):
            nxt = (k + 2) % 4
            wait_in(k)
            compute(k)
            issue_out(k)
            if k < 2:
                @pl.when(it > 0)
                def _():
                    wait_out(nxt)
            else:
                wait_out(nxt)

            @pl.when(j + k + 2 < NCHB)
            def _():
                issue_in(j + k + 2, nxt)
        return carry

    lax.fori_loop(0, _NB // 4, body, 0)
    wait_in(0)
    compute(0)
    issue_out(0)
    wait_out(2)
    wait_in(1)
    compute(1)
    issue_out(1)
    wait_out(3)
    wait_out(0)
    wait_out(1)
    plsc.subcore_barrier()
    pltpu.sync_copy(agg_s.at[pl.ds(r0, RPT)], agg_out.at[cid, pl.ds(r0, RPT)])

    @pl.when(sid == NS - 1)
    def _flush_tail():
        pltpu.sync_copy(agg_s.at[pl.ds(NS * RPT, N - NS * RPT)],
                        agg_out.at[cid, pl.ds(NS * RPT, N - NS * RPT)])


def _make_sb():
    return pl.kernel(
        _sb_body,
        out_type=[jax.ShapeDtypeStruct((NC, N, D), _F32)],
        mesh=_sc_mesh,
        scratch_types=[pltpu.VMEM((CB,), jnp.int32)] * 4
        + [pltpu.VMEM((CB, D), _F32)] * 4 + [
            pltpu.VMEM((2, D), _F32),
            pltpu.VMEM_SHARED((N, D), _F32),
        ] + [pltpu.SemaphoreType.DMA] * 8,
    )


def _prep_body(h, ws, wd, ps, pd):
    x = h[...]
    ps[...] = jnp.dot(x, ws[...], preferred_element_type=_F32)
    pd[...] = jnp.dot(x, wd[...], preferred_element_type=_F32)


def _prep(h, ws, wd):
    return pl.pallas_call(
        _prep_body,
        out_shape=[jax.ShapeDtypeStruct((N, D), _F32)] * 2,
    )(h, ws, wd)


_BE = 3200
_BE0 = 10000


def _t0_body(e, w, b, o):
    o[...] = jnp.dot(e[...], w[...], preferred_element_type=_F32) + b[...]


def _edge_t0(e, w, b):
    k = e.shape[1]
    return pl.pallas_call(
        _t0_body,
        grid=(E // _BE0,),
        in_specs=[
            pl.BlockSpec((_BE0, k), lambda i: (i, 0)),
            pl.BlockSpec((k, D), lambda i: (0, 0)),
            pl.BlockSpec((1, D), lambda i: (0, 0)),
        ],
        out_specs=pl.BlockSpec((_BE0, D), lambda i: (i, 0)),
        out_shape=jax.ShapeDtypeStruct((E, D), _F32),
    )(e, w, b)


def _t_body(epre, ss, w, b, o):
    e = jnp.maximum(epre[...] * ss[0:1, :] + ss[1:2, :], 0.0)
    o[...] = jnp.dot(e, w[...], preferred_element_type=_F32) + b[...]


def _edge_t(epre, ss, w, b):
    return pl.pallas_call(
        _t_body,
        grid=(E // _BE,),
        in_specs=[
            pl.BlockSpec((_BE, D), lambda i: (i, 0)),
            pl.BlockSpec((2, D), lambda i: (0, 0)),
            pl.BlockSpec((D, D), lambda i: (0, 0)),
            pl.BlockSpec((1, D), lambda i: (0, 0)),
        ],
        out_specs=pl.BlockSpec((_BE, D), lambda i: (i, 0)),
        out_shape=jax.ShapeDtypeStruct((E, D), _F32),
    )(epre, ss, w, b)


def _stats_body(parts, g, b, o):
    p = parts[:, 0, :]
    s = jnp.sum(p[:, :D], axis=0, keepdims=True) / E
    q = jnp.sum(p[:, D:], axis=0, keepdims=True) / E
    var = q - s * s
    scale = g[...] * lax.rsqrt(var + EPS)
    o[...] = jnp.concatenate([scale, b[...] - s * scale], axis=0)


def _stats_fin(parts, g, b):
    return pl.pallas_call(
        _stats_body,
        out_shape=jax.ShapeDtypeStruct((2, D), _F32),
    )(parts, g.reshape(1, D), b.reshape(1, D))


def _bn(x, g, b):
    m = jnp.mean(x, axis=0, keepdims=True)
    v = jnp.mean((x - m) * (x - m), axis=0, keepdims=True)
    return (x - m) * lax.rsqrt(v + EPS) * g + b


def _node_core(h_ref, agg_ref, w):
    agg = agg_ref[0] + agg_ref[1]
    x = (jnp.dot(h_ref[...], w["wg1a"][...], preferred_element_type=_F32)
         + jnp.dot(agg, w["wg1b"][...], preferred_element_type=_F32)
         + w["bg1"][...])
    x = jnp.maximum(_bn(x, w["g_g1"][...], w["b_g1"][...]), 0.0)
    x = jnp.dot(x, w["wg2"][...], preferred_element_type=_F32) + w["bg2"][...]
    x = jnp.maximum(_bn(x, w["g_g2"][...], w["b_g2"][...]), 0.0)
    x = _bn(x, w["g_nn"][...], w["b_nn"][...])
    return jnp.maximum(
        jnp.dot(x, w["wn"][...], preferred_element_type=_F32) + w["bn"][...], 0.0)


_MID_KEYS = ("wg1a", "wg1b", "bg1", "g_g1", "b_g1", "wg2", "bg2", "g_g2",
             "b_g2", "g_nn", "b_nn", "wn", "bn", "wes", "wed")


def _node_mid_body(h_ref, agg_ref, *refs):
    w = dict(zip(_MID_KEYS, refs[:len(_MID_KEYS)]))
    h_out, ps_out, pd_out = refs[len(_MID_KEYS):]
    hn = _node_core(h_ref, agg_ref, w)
    h_out[...] = hn
    ps_out[...] = jnp.dot(hn, w["wes"][...], preferred_element_type=_F32)
    pd_out[...] = jnp.dot(hn, w["wed"][...], preferred_element_type=_F32)


def _node_mid(h, agg, lw, we_next):
    args = lw + [we_next[:D], we_next[D:2 * D]]
    return pl.pallas_call(
        _node_mid_body,
        out_shape=[jax.ShapeDtypeStruct((N, D), _F32)] * 3,
    )(h, agg, *args)


_LAST_KEYS = _MID_KEYS[:13] + ("wp1", "bp1", "wp2", "bp2")


def _node_last_body(h_ref, agg_ref, *refs):
    w = dict(zip(_LAST_KEYS, refs[:len(_LAST_KEYS)]))
    out = refs[len(_LAST_KEYS)]
    hn = _node_core(h_ref, agg_ref, w)
    g = jnp.mean(hn, axis=0, keepdims=True)
    g = jnp.maximum(
        jnp.dot(g, w["wp1"][...], preferred_element_type=_F32) + w["bp1"][...], 0.0)
    out[...] = jnp.maximum(
        jnp.dot(g, w["wp2"][...], preferred_element_type=_F32) + w["bp2"][...], 0.0)


def _node_last(h, agg, lw, wp1, bp1, wp2, bp2):
    args = lw + [wp1, bp1.reshape(1, PROJ * 2), wp2, bp2.reshape(1, PROJ)]
    return pl.pallas_call(
        _node_last_body,
        out_shape=jax.ShapeDtypeStruct((1, PROJ), _F32),
    )(h, agg, *args)


def _layer_weights(p, i):
    wg1 = p["l%d_Wg1" % i]
    return [
        wg1[:D], wg1[D:], p["l%d_bg1" % i].reshape(1, D),
        p["l%d_g_g1" % i].reshape(1, D), p["l%d_b_g1" % i].reshape(1, D),
        p["l%d_Wg2" % i], p["l%d_bg2" % i].reshape(1, D),
        p["l%d_g_g2" % i].reshape(1, D), p["l%d_b_g2" % i].reshape(1, D),
        p["l%d_g_nn" % i].reshape(1, D), p["l%d_b_nn" % i].reshape(1, D),
        p["l%d_Wn" % i], p["l%d_bn" % i].reshape(1, D),
    ]


def kernel(node_attributes, edge_attributes, edge_indices, edge_indices_reverse, params):
    nodes = node_attributes[0]
    edges = edge_attributes[0]
    src = edge_indices[0, :, 0].reshape(NW, EW)
    dst = edge_indices[0, :, 1].reshape(NW, EW)
    agg_idx = edge_indices_reverse[0].reshape(NW, NCHB, CB)
    p = params

    sa = _make_sa()
    sb = _make_sb()

    h = nodes
    ps, pd = _prep(h, p["l0_We"][:D], p["l0_We"][D:2 * D])
    t = _edge_t0(edges, p["l0_We"][2 * D:], p["l0_be"].reshape(1, D))
    for i in range(3):
        epre, stats = sa(ps, pd, t, src, dst)
        ss = _stats_fin(stats, p["l%d_g_en" % i], p["l%d_b_en" % i])
        (agg,) = sb(epre, agg_idx, ss)
        if i < 2:
            we_n = p["l%d_We" % (i + 1)]
            t = _edge_t(epre, ss, we_n[2 * D:], p["l%d_be" % (i + 1)].reshape(1, D))
            h, ps, pd = _node_mid(h, agg, _layer_weights(p, i), we_n)
        else:
            emb = _node_last(h, agg, _layer_weights(p, i),
                             p["Wp1"], p["bp1"], p["Wp2"], p["bp2"])

    view = jnp.broadcast_to(emb[:, None, :], (1, 2, PROJ))
    return (emb, view)
